# baseline (speedup 1.0000x reference)
import functools

import jax
import jax.numpy as jnp
from jax.experimental import pallas as pl
from jax.experimental.pallas import tpu as pltpu

BATCH = 2
SEQ = 8192
HIDDEN = 1024
FFN = 3584
NUM_EXPERTS = 8
TOP_K = 2

T = BATCH * SEQ
LANES = 128


_RTM = 2048


def _router_body(x_ref, gw_ref, xbf_ref, idx_ref, wts_ref):
    x = x_ref[...]
    logits = jax.lax.dot_general(
        x, gw_ref[...], (((1,), (1,)), ((), ())),
        preferred_element_type=jnp.float32)
    lane = jax.lax.broadcasted_iota(jnp.int32, logits.shape, 1)
    neg = jnp.float32(-1e30)
    l0 = jnp.where(lane < NUM_EXPERTS, logits, neg)
    m1 = jnp.max(l0, axis=1, keepdims=True)
    i1 = jnp.min(jnp.where(l0 == m1, lane, LANES), axis=1, keepdims=True)
    l1 = jnp.where(lane == i1, neg, l0)
    m2 = jnp.max(l1, axis=1, keepdims=True)
    i2 = jnp.min(jnp.where(l1 == m2, lane, LANES), axis=1, keepdims=True)
    wa = 1.0 / (1.0 + jnp.exp(m2 - m1))
    wb = 1.0 - wa
    idx_ref[...] = jnp.where(lane == 0, i1, jnp.where(lane == 1, i2, 0))
    wts_ref[...] = jnp.where(lane == 0, wa, jnp.where(lane == 1, wb, 0.0))
    xbf_ref[...] = x.astype(jnp.bfloat16)


def _run_router(x, gate_wp):
    grid = (T // _RTM,)
    return pl.pallas_call(
        _router_body,
        grid=grid,
        in_specs=[
            pl.BlockSpec((_RTM, HIDDEN), lambda m: (m, 0)),
            pl.BlockSpec((LANES, HIDDEN), lambda m: (0, 0)),
        ],
        out_specs=[
            pl.BlockSpec((_RTM, HIDDEN), lambda m: (m, 0)),
            pl.BlockSpec((_RTM, LANES), lambda m: (m, 0)),
            pl.BlockSpec((_RTM, LANES), lambda m: (m, 0)),
        ],
        out_shape=[
            jax.ShapeDtypeStruct((T, HIDDEN), jnp.bfloat16),
            jax.ShapeDtypeStruct((T, LANES), jnp.int32),
            jax.ShapeDtypeStruct((T, LANES), jnp.float32),
        ],
    )(x, gate_wp)


_GTM = 512
_GTF = 512


def _n_tiles():
    return (TOP_K * T) // _GTM + NUM_EXPERTS


def _gmm_body(gid_ref, xs_ref, w1_ref, w3_ref, w2_ref, ys_ref):
    xb = xs_ref[...]
    acc = jnp.zeros((_GTM, HIDDEN), jnp.float32)
    for fj in range(FFN // _GTF):
        w1s = w1_ref[0, fj * _GTF:(fj + 1) * _GTF, :]
        w3s = w3_ref[0, fj * _GTF:(fj + 1) * _GTF, :]
        w2s = w2_ref[0, :, fj * _GTF:(fj + 1) * _GTF]
        a = jax.lax.dot_general(xb, w1s, (((1,), (1,)), ((), ())),
                                preferred_element_type=jnp.float32)
        b = jax.lax.dot_general(xb, w3s, (((1,), (1,)), ((), ())),
                                preferred_element_type=jnp.float32)
        h = (a * jax.nn.sigmoid(a) * b).astype(jnp.bfloat16)
        acc = acc + jax.lax.dot_general(h, w2s, (((1,), (1,)), ((), ())),
                                        preferred_element_type=jnp.float32)
    ys_ref[...] = acc


def _run_gmm(gid, xs, w1b, w3b, w2b):
    nt = _n_tiles()
    grid_spec = pltpu.PrefetchScalarGridSpec(
        num_scalar_prefetch=1,
        grid=(nt,),
        in_specs=[
            pl.BlockSpec((_GTM, HIDDEN), lambda m, gid_ref: (m, 0)),
            pl.BlockSpec((1, FFN, HIDDEN), lambda m, gid_ref: (gid_ref[m], 0, 0)),
            pl.BlockSpec((1, FFN, HIDDEN), lambda m, gid_ref: (gid_ref[m], 0, 0)),
            pl.BlockSpec((1, HIDDEN, FFN), lambda m, gid_ref: (gid_ref[m], 0, 0)),
        ],
        out_specs=pl.BlockSpec((_GTM, HIDDEN), lambda m, gid_ref: (m, 0)),
    )
    return pl.pallas_call(
        _gmm_body,
        grid_spec=grid_spec,
        out_shape=jax.ShapeDtypeStruct((nt * _GTM, HIDDEN), jnp.float32),
        compiler_params=pltpu.CompilerParams(
            dimension_semantics=("arbitrary",),
        ),
    )(gid, xs, w1b, w3b, w2b)


_SC_NC = 2
_SC_NS = 16
_SC_NW = _SC_NC * _SC_NS
_GCH = 128


def _run_sc_gather(xbf, sorted_tok):
    from jax.experimental.pallas import tpu_sc as plsc

    m_pad = _n_tiles() * _GTM
    rows_per_w = m_pad // _SC_NW
    n_chunks = rows_per_w // _GCH
    hw = HIDDEN // 2
    xi = jax.lax.bitcast_convert_type(
        xbf.reshape(T, hw, 2), jnp.int32)
    mesh = plsc.VectorSubcoreMesh(core_axis_name="c", subcore_axis_name="s")

    @functools.partial(
        pl.kernel, mesh=mesh,
        out_type=jax.ShapeDtypeStruct((m_pad, hw), jnp.int32),
        scratch_types=[
            pltpu.VMEM((_GCH,), jnp.int32),
            pltpu.VMEM((_GCH, hw), jnp.int32),
            pltpu.SemaphoreType.DMA,
        ],
    )
    def gather_kernel(x_hbm, idx_hbm, out_hbm, idx_v, rows_v, sem):
        wid = jax.lax.axis_index("s") * _SC_NC + jax.lax.axis_index("c")
        base = wid * rows_per_w

        def body(c, carry):
            off = base + c * _GCH
            pltpu.sync_copy(idx_hbm.at[pl.ds(off, _GCH)], idx_v)
            pltpu.async_copy(x_hbm.at[idx_v], rows_v, sem).wait()
            pltpu.sync_copy(rows_v, out_hbm.at[pl.ds(off, _GCH)])
            return carry

        jax.lax.fori_loop(0, n_chunks, body, 0)

    xsi = gather_kernel(xi, sorted_tok)
    return jax.lax.bitcast_convert_type(
        xsi[..., None], jnp.bfloat16).reshape(m_pad, HIDDEN)


def kernel(hidden_states, gate_w, w1, w2, w3):
    x = hidden_states.reshape(-1, HIDDEN)
    gate_wp = jnp.zeros((LANES, HIDDEN), jnp.float32).at[:NUM_EXPERTS].set(gate_w)
    xbf, idx_out, wts_out = _run_router(x, gate_wp)
    idx2 = idx_out[:, :TOP_K]
    wts2 = wts_out[:, :TOP_K]

    s = TOP_K * T
    m_pad = _n_tiles() * _GTM
    eid = idx2.reshape(s)
    order = jnp.argsort(eid, stable=True)
    sorted_eid = eid[order]
    counts = jnp.bincount(eid, length=NUM_EXPERTS)
    start = jnp.concatenate([jnp.zeros((1,), counts.dtype), jnp.cumsum(counts)[:-1]])
    pcnt = ((counts + _GTM - 1) // _GTM) * _GTM
    poff = jnp.concatenate([jnp.zeros((1,), pcnt.dtype), jnp.cumsum(pcnt)[:-1]])
    rank = jnp.arange(s) - start[sorted_eid]
    dest = (poff[sorted_eid] + rank).astype(jnp.int32)
    sorted_tok = jnp.zeros((m_pad,), jnp.int32).at[dest].set(
        (order // TOP_K).astype(jnp.int32))
    inv = jnp.zeros((s,), jnp.int32).at[order].set(dest)
    tile_start = jnp.arange(_n_tiles()) * _GTM
    poff_end = jnp.cumsum(pcnt)
    gid = jnp.clip(jnp.sum(tile_start[:, None] >= poff_end[None, :], axis=1),
                   0, NUM_EXPERTS - 1).astype(jnp.int32)

    xs = _run_sc_gather(xbf, sorted_tok)
    w1b = w1.astype(jnp.bfloat16)
    w3b = w3.astype(jnp.bfloat16)
    w2b = w2.astype(jnp.bfloat16)
    ys = _run_gmm(gid, xs, w1b, w3b, w2b)

    pos_a = inv[0::TOP_K]
    pos_b = inv[1::TOP_K]
    final = wts2[:, 0:1] * ys[pos_a] + wts2[:, 1:2] * ys[pos_b]
    return final.reshape(BATCH, SEQ, HIDDEN)


# speedup vs baseline: 1.4487x; 1.4487x over previous
import functools

import jax
import jax.numpy as jnp
from jax.experimental import pallas as pl
from jax.experimental.pallas import tpu as pltpu

BATCH = 2
SEQ = 8192
HIDDEN = 1024
FFN = 3584
NUM_EXPERTS = 8
TOP_K = 2

T = BATCH * SEQ
LANES = 128


_RTM = 2048


def _router_body(x_ref, gw_ref, idx_ref, wts_ref):
    x = x_ref[...]
    logits = jax.lax.dot_general(
        x, gw_ref[...], (((1,), (1,)), ((), ())),
        preferred_element_type=jnp.float32)
    lane = jax.lax.broadcasted_iota(jnp.int32, logits.shape, 1)
    neg = jnp.float32(-1e30)
    l0 = jnp.where(lane < NUM_EXPERTS, logits, neg)
    m1 = jnp.max(l0, axis=1, keepdims=True)
    i1 = jnp.min(jnp.where(l0 == m1, lane, LANES), axis=1, keepdims=True)
    l1 = jnp.where(lane == i1, neg, l0)
    m2 = jnp.max(l1, axis=1, keepdims=True)
    i2 = jnp.min(jnp.where(l1 == m2, lane, LANES), axis=1, keepdims=True)
    wa = 1.0 / (1.0 + jnp.exp(m2 - m1))
    wb = 1.0 - wa
    idx_ref[...] = jnp.where(lane == 0, i1, jnp.where(lane == 1, i2, 0))
    wts_ref[...] = jnp.where(lane == 0, wa, jnp.where(lane == 1, wb, 0.0))


def _run_router(x, gate_wp):
    grid = (T // _RTM,)
    return pl.pallas_call(
        _router_body,
        grid=grid,
        in_specs=[
            pl.BlockSpec((_RTM, HIDDEN), lambda m: (m, 0)),
            pl.BlockSpec((LANES, HIDDEN), lambda m: (0, 0)),
        ],
        out_specs=[
            pl.BlockSpec((_RTM, LANES), lambda m: (m, 0)),
            pl.BlockSpec((_RTM, LANES), lambda m: (m, 0)),
        ],
        out_shape=[
            jax.ShapeDtypeStruct((T, LANES), jnp.int32),
            jax.ShapeDtypeStruct((T, LANES), jnp.float32),
        ],
    )(x, gate_wp)


_GTM = 512
_GTF = 512


def _n_tiles():
    return (TOP_K * T) // _GTM + NUM_EXPERTS


def _gmm_body(gid_ref, xs_ref, w1_ref, w3_ref, w2_ref, ys_ref):
    xb = xs_ref[...].astype(jnp.bfloat16)
    acc = jnp.zeros((_GTM, HIDDEN), jnp.float32)
    for fj in range(FFN // _GTF):
        w1s = w1_ref[0, fj * _GTF:(fj + 1) * _GTF, :]
        w3s = w3_ref[0, fj * _GTF:(fj + 1) * _GTF, :]
        w2s = w2_ref[0, :, fj * _GTF:(fj + 1) * _GTF]
        a = jax.lax.dot_general(xb, w1s, (((1,), (1,)), ((), ())),
                                preferred_element_type=jnp.float32)
        b = jax.lax.dot_general(xb, w3s, (((1,), (1,)), ((), ())),
                                preferred_element_type=jnp.float32)
        h = (a * jax.nn.sigmoid(a) * b).astype(jnp.bfloat16)
        acc = acc + jax.lax.dot_general(h, w2s, (((1,), (1,)), ((), ())),
                                        preferred_element_type=jnp.float32)
    ys_ref[...] = acc


def _run_gmm(gid, xs, w1b, w3b, w2b):
    nt = _n_tiles()
    grid_spec = pltpu.PrefetchScalarGridSpec(
        num_scalar_prefetch=1,
        grid=(nt,),
        in_specs=[
            pl.BlockSpec((_GTM, HIDDEN), lambda m, gid_ref: (m, 0)),
            pl.BlockSpec((1, FFN, HIDDEN), lambda m, gid_ref: (gid_ref[m], 0, 0)),
            pl.BlockSpec((1, FFN, HIDDEN), lambda m, gid_ref: (gid_ref[m], 0, 0)),
            pl.BlockSpec((1, HIDDEN, FFN), lambda m, gid_ref: (gid_ref[m], 0, 0)),
        ],
        out_specs=pl.BlockSpec((_GTM, HIDDEN), lambda m, gid_ref: (m, 0)),
    )
    return pl.pallas_call(
        _gmm_body,
        grid_spec=grid_spec,
        out_shape=jax.ShapeDtypeStruct((nt * _GTM, HIDDEN), jnp.float32),
        compiler_params=pltpu.CompilerParams(
            dimension_semantics=("arbitrary",),
        ),
    )(gid, xs, w1b, w3b, w2b)


_SC_NC = 2
_SC_NS = 16
_SC_NW = _SC_NC * _SC_NS
_GCH = 48


def _run_sc_gather(x, sorted_tok):
    from jax.experimental.pallas import tpu_sc as plsc

    m_pad = _n_tiles() * _GTM
    rows_per_w = m_pad // _SC_NW
    n_chunks = rows_per_w // _GCH
    assert n_chunks % 2 == 0
    mesh = plsc.VectorSubcoreMesh(core_axis_name="c", subcore_axis_name="s")

    @functools.partial(
        pl.kernel, mesh=mesh,
        out_type=jax.ShapeDtypeStruct((m_pad, HIDDEN), jnp.float32),
        scratch_types=[
            pltpu.VMEM((rows_per_w,), jnp.int32),
            pltpu.VMEM((_GCH, HIDDEN), jnp.float32),
            pltpu.VMEM((_GCH, HIDDEN), jnp.float32),
            pltpu.SemaphoreType.DMA,
        ],
    )
    def gather_kernel(x_hbm, idx_hbm, out_hbm, idx_v, buf0, buf1, gsem):
        wid = jax.lax.axis_index("s") * _SC_NC + jax.lax.axis_index("c")
        base = wid * rows_per_w
        pltpu.sync_copy(idx_hbm.at[pl.ds(base, rows_per_w)], idx_v)
        pltpu.async_copy(x_hbm.at[idx_v.at[pl.ds(0, _GCH)]], buf0, gsem)

        def pair(j, carry):
            c0 = 2 * j
            pltpu.make_async_copy(
                x_hbm.at[idx_v.at[pl.ds(0, _GCH)]], buf0, gsem).wait()

            @pl.when(c0 + 1 < n_chunks)
            def _():
                pltpu.async_copy(
                    x_hbm.at[idx_v.at[pl.ds((c0 + 1) * _GCH, _GCH)]],
                    buf1, gsem)

            pltpu.sync_copy(buf0, out_hbm.at[pl.ds(base + c0 * _GCH, _GCH)])

            pltpu.make_async_copy(
                x_hbm.at[idx_v.at[pl.ds(0, _GCH)]], buf1, gsem).wait()

            @pl.when(c0 + 2 < n_chunks)
            def _():
                pltpu.async_copy(
                    x_hbm.at[idx_v.at[pl.ds((c0 + 2) * _GCH, _GCH)]],
                    buf0, gsem)

            pltpu.sync_copy(
                buf1, out_hbm.at[pl.ds(base + (c0 + 1) * _GCH, _GCH)])
            return carry

        jax.lax.fori_loop(0, n_chunks // 2, pair, 0)

    return gather_kernel(x, sorted_tok)


def kernel(hidden_states, gate_w, w1, w2, w3):
    x = hidden_states.reshape(-1, HIDDEN)
    gate_wp = jnp.zeros((LANES, HIDDEN), jnp.float32).at[:NUM_EXPERTS].set(gate_w)
    idx_out, wts_out = _run_router(x, gate_wp)
    idx2 = idx_out[:, :TOP_K]
    wts2 = wts_out[:, :TOP_K]

    s = TOP_K * T
    m_pad = _n_tiles() * _GTM
    eid = idx2.reshape(s)
    order = jnp.argsort(eid, stable=True)
    sorted_eid = eid[order]
    counts = jnp.bincount(eid, length=NUM_EXPERTS)
    start = jnp.concatenate([jnp.zeros((1,), counts.dtype), jnp.cumsum(counts)[:-1]])
    pcnt = ((counts + _GTM - 1) // _GTM) * _GTM
    poff = jnp.concatenate([jnp.zeros((1,), pcnt.dtype), jnp.cumsum(pcnt)[:-1]])
    rank = jnp.arange(s) - start[sorted_eid]
    dest = (poff[sorted_eid] + rank).astype(jnp.int32)
    sorted_tok = jnp.zeros((m_pad,), jnp.int32).at[dest].set(
        (order // TOP_K).astype(jnp.int32))
    inv = jnp.zeros((s,), jnp.int32).at[order].set(dest)
    tile_start = jnp.arange(_n_tiles()) * _GTM
    poff_end = jnp.cumsum(pcnt)
    gid = jnp.clip(jnp.sum(tile_start[:, None] >= poff_end[None, :], axis=1),
                   0, NUM_EXPERTS - 1).astype(jnp.int32)

    xs = _run_sc_gather(x, sorted_tok)
    w1b = w1.astype(jnp.bfloat16)
    w3b = w3.astype(jnp.bfloat16)
    w2b = w2.astype(jnp.bfloat16)
    ys = _run_gmm(gid, xs, w1b, w3b, w2b)

    pos_a = inv[0::TOP_K]
    pos_b = inv[1::TOP_K]
    final = wts2[:, 0:1] * ys[pos_a] + wts2[:, 1:2] * ys[pos_b]
    return final.reshape(BATCH, SEQ, HIDDEN)


# speedup vs baseline: 1.5451x; 1.0665x over previous
import functools

import jax
import jax.numpy as jnp
from jax.experimental import pallas as pl
from jax.experimental.pallas import tpu as pltpu

BATCH = 2
SEQ = 8192
HIDDEN = 1024
FFN = 3584
NUM_EXPERTS = 8
TOP_K = 2

T = BATCH * SEQ
LANES = 128


_RTM = 2048


def _router_body(x_ref, gw_ref, idx_ref, wts_ref, xi_ref):
    x = x_ref[...]
    logits = jax.lax.dot_general(
        x, gw_ref[...], (((1,), (1,)), ((), ())),
        preferred_element_type=jnp.float32)
    lane = jax.lax.broadcasted_iota(jnp.int32, logits.shape, 1)
    neg = jnp.float32(-1e30)
    l0 = jnp.where(lane < NUM_EXPERTS, logits, neg)
    m1 = jnp.max(l0, axis=1, keepdims=True)
    i1 = jnp.min(jnp.where(l0 == m1, lane, LANES), axis=1, keepdims=True)
    l1 = jnp.where(lane == i1, neg, l0)
    m2 = jnp.max(l1, axis=1, keepdims=True)
    i2 = jnp.min(jnp.where(l1 == m2, lane, LANES), axis=1, keepdims=True)
    wa = 1.0 / (1.0 + jnp.exp(m2 - m1))
    wb = 1.0 - wa
    idx_ref[...] = jnp.where(lane == 0, i1, jnp.where(lane == 1, i2, 0))
    wts_ref[...] = jnp.where(lane == 0, wa, jnp.where(lane == 1, wb, 0.0))
    xu = jax.lax.bitcast_convert_type(x, jnp.uint32)
    rb = (xu + jnp.uint32(0x7FFF) + ((xu >> 16) & jnp.uint32(1))) >> 16
    hw = HIDDEN // 2
    xi = rb[:, :hw] | (rb[:, hw:] << 16)
    xi_ref[...] = jax.lax.bitcast_convert_type(xi, jnp.int32)


def _run_router(x, gate_wp):
    grid = (T // _RTM,)
    return pl.pallas_call(
        _router_body,
        grid=grid,
        in_specs=[
            pl.BlockSpec((_RTM, HIDDEN), lambda m: (m, 0)),
            pl.BlockSpec((LANES, HIDDEN), lambda m: (0, 0)),
        ],
        out_specs=[
            pl.BlockSpec((_RTM, LANES), lambda m: (m, 0)),
            pl.BlockSpec((_RTM, LANES), lambda m: (m, 0)),
            pl.BlockSpec((_RTM, HIDDEN // 2), lambda m: (m, 0)),
        ],
        out_shape=[
            jax.ShapeDtypeStruct((T, LANES), jnp.int32),
            jax.ShapeDtypeStruct((T, LANES), jnp.float32),
            jax.ShapeDtypeStruct((T, HIDDEN // 2), jnp.int32),
        ],
    )(x, gate_wp)


_GTM = 512
_GTF = 512


def _n_tiles():
    return (TOP_K * T) // _GTM + NUM_EXPERTS


def _gmm_body(gid_ref, xs_ref, w1_ref, w3_ref, w2_ref, ys_ref):
    xu = jax.lax.bitcast_convert_type(xs_ref[...], jnp.uint32)
    xlo = jax.lax.bitcast_convert_type(xu << 16, jnp.float32)
    xhi = jax.lax.bitcast_convert_type(xu & jnp.uint32(0xFFFF0000), jnp.float32)
    xb = jnp.concatenate([xlo.astype(jnp.bfloat16),
                          xhi.astype(jnp.bfloat16)], axis=1)
    acc = jnp.zeros((_GTM, HIDDEN), jnp.float32)
    for fj in range(FFN // _GTF):
        w1s = w1_ref[0, fj * _GTF:(fj + 1) * _GTF, :]
        w3s = w3_ref[0, fj * _GTF:(fj + 1) * _GTF, :]
        w2s = w2_ref[0, :, fj * _GTF:(fj + 1) * _GTF]
        a = jax.lax.dot_general(xb, w1s, (((1,), (1,)), ((), ())),
                                preferred_element_type=jnp.float32)
        b = jax.lax.dot_general(xb, w3s, (((1,), (1,)), ((), ())),
                                preferred_element_type=jnp.float32)
        h = (a * jax.nn.sigmoid(a) * b).astype(jnp.bfloat16)
        acc = acc + jax.lax.dot_general(h, w2s, (((1,), (1,)), ((), ())),
                                        preferred_element_type=jnp.float32)
    ys_ref[...] = acc


def _run_gmm(gid, xs, w1b, w3b, w2b):
    nt = _n_tiles()
    grid_spec = pltpu.PrefetchScalarGridSpec(
        num_scalar_prefetch=1,
        grid=(nt,),
        in_specs=[
            pl.BlockSpec((_GTM, HIDDEN // 2), lambda m, gid_ref: (m, 0)),
            pl.BlockSpec((1, FFN, HIDDEN), lambda m, gid_ref: (gid_ref[m], 0, 0)),
            pl.BlockSpec((1, FFN, HIDDEN), lambda m, gid_ref: (gid_ref[m], 0, 0)),
            pl.BlockSpec((1, HIDDEN, FFN), lambda m, gid_ref: (gid_ref[m], 0, 0)),
        ],
        out_specs=pl.BlockSpec((_GTM, HIDDEN), lambda m, gid_ref: (m, 0)),
    )
    return pl.pallas_call(
        _gmm_body,
        grid_spec=grid_spec,
        out_shape=jax.ShapeDtypeStruct((nt * _GTM, HIDDEN), jnp.float32),
        compiler_params=pltpu.CompilerParams(
            dimension_semantics=("arbitrary",),
        ),
    )(gid, xs, w1b, w3b, w2b)


_SC_NC = 2
_SC_NS = 16
_SC_NW = _SC_NC * _SC_NS
_GCH = 48


def _run_sc_gather(xi, sorted_tok):
    from jax.experimental.pallas import tpu_sc as plsc

    m_pad = _n_tiles() * _GTM
    hw = HIDDEN // 2
    rows_per_w = m_pad // _SC_NW
    n_chunks = rows_per_w // _GCH
    assert n_chunks % 4 == 0 and n_chunks >= 8
    idx3 = sorted_tok.reshape(_SC_NW, n_chunks, _GCH)
    mesh = plsc.VectorSubcoreMesh(core_axis_name="c", subcore_axis_name="s")

    @functools.partial(
        pl.kernel, mesh=mesh,
        out_type=jax.ShapeDtypeStruct((m_pad, hw), jnp.int32),
        scratch_types=[
            pltpu.VMEM((n_chunks, _GCH), jnp.int32),
            [pltpu.VMEM((_GCH, hw), jnp.int32)] * 4,
            [pltpu.SemaphoreType.DMA] * 4,
            [pltpu.SemaphoreType.DMA] * 4,
        ],
    )
    def gather_kernel(x_hbm, idx_hbm, out_hbm, idx_all, bufs, gsems, wsems):
        wid = jax.lax.axis_index("s") * _SC_NC + jax.lax.axis_index("c")
        base = wid * rows_per_w
        pltpu.sync_copy(idx_hbm.at[wid], idx_all)

        def fire_gather(c, u):
            pltpu.async_copy(x_hbm.at[idx_all.at[c]], bufs[u], gsems[u])

        def wait_gather(u):
            pltpu.make_async_copy(
                x_hbm.at[idx_all.at[0]], bufs[u], gsems[u]).wait()

        def fire_wb(c, u):
            pltpu.async_copy(
                bufs[u], out_hbm.at[pl.ds(base + c * _GCH, _GCH)], wsems[u])

        def wait_wb(u):
            pltpu.make_async_copy(
                bufs[u], out_hbm.at[pl.ds(base, _GCH)], wsems[u]).wait()

        fire_gather(0, 0)
        fire_gather(1, 1)
        fire_gather(2, 2)

        def quad(j, carry):
            for u in range(4):
                c = 4 * j + u
                wait_gather(u)
                fire_wb(c, u)
                nxt = (u + 3) % 4

                @pl.when((c >= 1) & (c + 3 < n_chunks))
                def _():
                    wait_wb(nxt)

                @pl.when(c + 3 < n_chunks)
                def _():
                    fire_gather(c + 3, nxt)
            return carry

        jax.lax.fori_loop(0, n_chunks // 4, quad, 0)
        for u in range(4):
            wait_wb(u)

    xsi = gather_kernel(xi, idx3)
    return xsi


def kernel(hidden_states, gate_w, w1, w2, w3):
    x = hidden_states.reshape(-1, HIDDEN)
    gate_wp = jnp.zeros((LANES, HIDDEN), jnp.float32).at[:NUM_EXPERTS].set(gate_w)
    idx_out, wts_out, xi = _run_router(x, gate_wp)
    idx2 = idx_out[:, :TOP_K]
    wts2 = wts_out[:, :TOP_K]

    s = TOP_K * T
    m_pad = _n_tiles() * _GTM
    eid = idx2.reshape(s)
    order = jnp.argsort(eid, stable=True)
    sorted_eid = eid[order]
    counts = jnp.bincount(eid, length=NUM_EXPERTS)
    start = jnp.concatenate([jnp.zeros((1,), counts.dtype), jnp.cumsum(counts)[:-1]])
    pcnt = ((counts + _GTM - 1) // _GTM) * _GTM
    poff = jnp.concatenate([jnp.zeros((1,), pcnt.dtype), jnp.cumsum(pcnt)[:-1]])
    rank = jnp.arange(s) - start[sorted_eid]
    dest = (poff[sorted_eid] + rank).astype(jnp.int32)
    sorted_tok = jnp.zeros((m_pad,), jnp.int32).at[dest].set(
        (order // TOP_K).astype(jnp.int32))
    inv = jnp.zeros((s,), jnp.int32).at[order].set(dest)
    tile_start = jnp.arange(_n_tiles()) * _GTM
    poff_end = jnp.cumsum(pcnt)
    gid = jnp.clip(jnp.sum(tile_start[:, None] >= poff_end[None, :], axis=1),
                   0, NUM_EXPERTS - 1).astype(jnp.int32)

    xs = _run_sc_gather(xi, sorted_tok)
    w1b = w1.astype(jnp.bfloat16)
    w3b = w3.astype(jnp.bfloat16)
    w2b = w2.astype(jnp.bfloat16)
    ys = _run_gmm(gid, xs, w1b, w3b, w2b)

    pos_a = inv[0::TOP_K]
    pos_b = inv[1::TOP_K]
    final = wts2[:, 0:1] * ys[pos_a] + wts2[:, 1:2] * ys[pos_b]
    return final.reshape(BATCH, SEQ, HIDDEN)


# speedup vs baseline: 1.7628x; 1.1409x over previous
import functools

import jax
import jax.numpy as jnp
from jax.experimental import pallas as pl
from jax.experimental.pallas import tpu as pltpu

BATCH = 2
SEQ = 8192
HIDDEN = 1024
FFN = 3584
NUM_EXPERTS = 8
TOP_K = 2

T = BATCH * SEQ
LANES = 128


_RTM = 2048
_WTOK = 512
_HTOK = 32


def _router_body(x_ref, gw_ref, idx_ref, wts_ref, xi_ref, hist_ref):
    x = x_ref[...]
    logits = jax.lax.dot_general(
        x, gw_ref[...], (((1,), (1,)), ((), ())),
        preferred_element_type=jnp.float32)
    lane = jax.lax.broadcasted_iota(jnp.int32, logits.shape, 1)
    neg = jnp.float32(-1e30)
    l0 = jnp.where(lane < NUM_EXPERTS, logits, neg)
    m1 = jnp.max(l0, axis=1, keepdims=True)
    i1 = jnp.min(jnp.where(l0 == m1, lane, LANES), axis=1, keepdims=True)
    l1 = jnp.where(lane == i1, neg, l0)
    m2 = jnp.max(l1, axis=1, keepdims=True)
    i2 = jnp.min(jnp.where(l1 == m2, lane, LANES), axis=1, keepdims=True)
    wa = 1.0 / (1.0 + jnp.exp(m2 - m1))
    wb = 1.0 - wa
    idx_ref[...] = jnp.where(lane == 0, i1, jnp.where(lane == 1, i2, 0))
    wts_ref[...] = jnp.where(lane == 0, wa, jnp.where(lane == 1, wb, 0.0))
    xu = jax.lax.bitcast_convert_type(x, jnp.uint32)
    rb = (xu + jnp.uint32(0x7FFF) + ((xu >> 16) & jnp.uint32(1))) >> 16
    hw = HIDDEN // 2
    xi = rb[:, :hw] | (rb[:, hw:] << 16)
    xi_ref[...] = jax.lax.bitcast_convert_type(xi, jnp.int32)
    oh = (lane == i1).astype(jnp.float32) + (lane == i2).astype(jnp.float32)
    sub = _RTM // _HTOK
    row = jax.lax.broadcasted_iota(jnp.int32, (_RTM, sub), 0) // _HTOK
    col = jax.lax.broadcasted_iota(jnp.int32, (_RTM, sub), 1)
    sel = (row == col).astype(jnp.float32)
    hs = jax.lax.dot_general(sel, oh, (((0,), (0,)), ((), ())),
                             preferred_element_type=jnp.float32)
    hist_ref[...] = hs.astype(jnp.int32)[None]


def _run_router(x, gate_wp):
    grid = (T // _RTM,)
    return pl.pallas_call(
        _router_body,
        grid=grid,
        in_specs=[
            pl.BlockSpec((_RTM, HIDDEN), lambda m: (m, 0)),
            pl.BlockSpec((LANES, HIDDEN), lambda m: (0, 0)),
        ],
        out_specs=[
            pl.BlockSpec((_RTM, LANES), lambda m: (m, 0)),
            pl.BlockSpec((_RTM, LANES), lambda m: (m, 0)),
            pl.BlockSpec((_RTM, HIDDEN // 2), lambda m: (m, 0)),
            pl.BlockSpec((1, _RTM // _HTOK, LANES), lambda m: (m, 0, 0)),
        ],
        out_shape=[
            jax.ShapeDtypeStruct((T, LANES), jnp.int32),
            jax.ShapeDtypeStruct((T, LANES), jnp.float32),
            jax.ShapeDtypeStruct((T, HIDDEN // 2), jnp.int32),
            jax.ShapeDtypeStruct((T // _RTM, _RTM // _HTOK, LANES), jnp.int32),
        ],
    )(x, gate_wp)


_GTM = 512
_GTF = 512


def _n_tiles():
    return (TOP_K * T) // _GTM + NUM_EXPERTS


def _gmm_body(gid_ref, xs_ref, w1_ref, w3_ref, w2_ref, ys_ref):
    xu = jax.lax.bitcast_convert_type(xs_ref[...], jnp.uint32)
    xlo = jax.lax.bitcast_convert_type(xu << 16, jnp.float32)
    xhi = jax.lax.bitcast_convert_type(xu & jnp.uint32(0xFFFF0000), jnp.float32)
    xb = jnp.concatenate([xlo.astype(jnp.bfloat16),
                          xhi.astype(jnp.bfloat16)], axis=1)
    acc = jnp.zeros((_GTM, HIDDEN), jnp.float32)
    for fj in range(FFN // _GTF):
        w1s = w1_ref[0, fj * _GTF:(fj + 1) * _GTF, :]
        w3s = w3_ref[0, fj * _GTF:(fj + 1) * _GTF, :]
        w2s = w2_ref[0, :, fj * _GTF:(fj + 1) * _GTF]
        a = jax.lax.dot_general(xb, w1s, (((1,), (1,)), ((), ())),
                                preferred_element_type=jnp.float32)
        b = jax.lax.dot_general(xb, w3s, (((1,), (1,)), ((), ())),
                                preferred_element_type=jnp.float32)
        h = (a * jax.nn.sigmoid(a) * b).astype(jnp.bfloat16)
        acc = acc + jax.lax.dot_general(h, w2s, (((1,), (1,)), ((), ())),
                                        preferred_element_type=jnp.float32)
    ys_ref[...] = acc


def _run_gmm(gid, xs, w1b, w3b, w2b):
    nt = _n_tiles()
    grid_spec = pltpu.PrefetchScalarGridSpec(
        num_scalar_prefetch=1,
        grid=(nt,),
        in_specs=[
            pl.BlockSpec((_GTM, HIDDEN // 2), lambda m, gid_ref: (m, 0)),
            pl.BlockSpec((1, FFN, HIDDEN), lambda m, gid_ref: (gid_ref[m], 0, 0)),
            pl.BlockSpec((1, FFN, HIDDEN), lambda m, gid_ref: (gid_ref[m], 0, 0)),
            pl.BlockSpec((1, HIDDEN, FFN), lambda m, gid_ref: (gid_ref[m], 0, 0)),
        ],
        out_specs=pl.BlockSpec((_GTM, HIDDEN), lambda m, gid_ref: (m, 0)),
    )
    return pl.pallas_call(
        _gmm_body,
        grid_spec=grid_spec,
        out_shape=jax.ShapeDtypeStruct((nt * _GTM, HIDDEN), jnp.float32),
        compiler_params=pltpu.CompilerParams(
            dimension_semantics=("arbitrary",),
        ),
    )(gid, xs, w1b, w3b, w2b)


_SLOTS_W = 2 * _WTOK


def _run_sc_binning(eid_t, info):
    from jax.experimental.pallas import tpu_sc as plsc

    m_pad = _n_tiles() * _GTM
    mesh = plsc.VectorSubcoreMesh(core_axis_name="c", subcore_axis_name="s")

    @functools.partial(
        pl.kernel, mesh=mesh,
        out_type=[
            jax.ShapeDtypeStruct((m_pad,), jnp.int32),
            jax.ShapeDtypeStruct((_SC_NW, _SLOTS_W), jnp.int32),
        ],
        scratch_types=[
            pltpu.VMEM((_SLOTS_W,), jnp.int32),
            pltpu.VMEM((NUM_EXPERTS, LANES), jnp.int32),
            pltpu.VMEM((_SLOTS_W,), jnp.int32),
            pltpu.VMEM((_SLOTS_W,), jnp.int32),
            pltpu.VMEM((_SLOTS_W // LANES, LANES), jnp.int32),
            pltpu.VMEM((_SLOTS_W // LANES, LANES), jnp.int32),
            pltpu.SemaphoreType.DMA,
        ],
    )
    def binning_kernel(eid_hbm, info_hbm, stok_hbm, inv_hbm,
                       eid_v, info_v, dest1, tok1, dest2, tok2, sem):
        wid = jax.lax.axis_index("s") * _SC_NC + jax.lax.axis_index("c")
        pltpu.sync_copy(eid_hbm.at[wid], eid_v)
        pltpu.sync_copy(info_hbm.at[wid], info_v)
        lane16 = jax.lax.broadcasted_iota(jnp.int32, (16,), 0)
        n_it = _SLOTS_W // 16

        cnts = [info_v[e, pl.ds(0, 16)] for e in range(NUM_EXPERTS)]
        for i in range(n_it):
            ids = eid_v[pl.ds(i * 16, 16)]
            dest = jnp.zeros((16,), jnp.int32)
            for en in range(NUM_EXPERTS):
                one = 1 - jnp.minimum(jnp.abs(ids - en), 1)
                dest = dest + one * cnts[en]
                cnts[en] = cnts[en] + one
            dest1[pl.ds(i * 16, 16)] = dest
            tok1[pl.ds(i * 16, 16)] = (
                wid * _SLOTS_W + lane16 * n_it + i) >> 1
        pltpu.sync_copy(dest1, inv_hbm.at[wid])

        for r in range(_SLOTS_W // LANES):
            for m in range(LANES // 16):
                p = r * LANES + m * 16
                dest2[r, pl.ds(m * 16, 16)] = dest1[pl.ds(p, 16)]
                tok2[r, pl.ds(m * 16, 16)] = tok1[pl.ds(p, 16)]
        for r in range(_SLOTS_W // LANES):
            pltpu.async_copy(tok2.at[r], stok_hbm.at[dest2.at[r]], sem)
        for r in range(_SLOTS_W // LANES):
            pltpu.make_async_copy(tok2.at[0], stok_hbm.at[dest2.at[0]],
                                  sem).wait()

    return binning_kernel(eid_t, info)


_SC_NC = 2
_SC_NS = 16
_SC_NW = _SC_NC * _SC_NS
_GCH = 48


def _run_sc_gather(xi, sorted_tok):
    from jax.experimental.pallas import tpu_sc as plsc

    m_pad = _n_tiles() * _GTM
    hw = HIDDEN // 2
    rows_per_w = m_pad // _SC_NW
    n_chunks = rows_per_w // _GCH
    assert n_chunks % 4 == 0 and n_chunks >= 8
    idx3 = sorted_tok.reshape(_SC_NW, n_chunks, _GCH)
    mesh = plsc.VectorSubcoreMesh(core_axis_name="c", subcore_axis_name="s")

    @functools.partial(
        pl.kernel, mesh=mesh,
        out_type=jax.ShapeDtypeStruct((m_pad, hw), jnp.int32),
        scratch_types=[
            pltpu.VMEM((n_chunks, _GCH), jnp.int32),
            [pltpu.VMEM((_GCH, hw), jnp.int32)] * 4,
            [pltpu.SemaphoreType.DMA] * 4,
            [pltpu.SemaphoreType.DMA] * 4,
        ],
    )
    def gather_kernel(x_hbm, idx_hbm, out_hbm, idx_all, bufs, gsems, wsems):
        wid = jax.lax.axis_index("s") * _SC_NC + jax.lax.axis_index("c")
        base = wid * rows_per_w
        pltpu.sync_copy(idx_hbm.at[wid], idx_all)

        def clamp(c, carry):
            for m in range(_GCH // 16):
                v = idx_all[c, pl.ds(m * 16, 16)]
                idx_all[c, pl.ds(m * 16, 16)] = jnp.clip(v, 0, T - 1)
            return carry

        jax.lax.fori_loop(0, n_chunks, clamp, 0)

        def fire_gather(c, u):
            pltpu.async_copy(x_hbm.at[idx_all.at[c]], bufs[u], gsems[u])

        def wait_gather(u):
            pltpu.make_async_copy(
                x_hbm.at[idx_all.at[0]], bufs[u], gsems[u]).wait()

        def fire_wb(c, u):
            pltpu.async_copy(
                bufs[u], out_hbm.at[pl.ds(base + c * _GCH, _GCH)], wsems[u])

        def wait_wb(u):
            pltpu.make_async_copy(
                bufs[u], out_hbm.at[pl.ds(base, _GCH)], wsems[u]).wait()

        fire_gather(0, 0)
        fire_gather(1, 1)
        fire_gather(2, 2)

        def quad(j, carry):
            for u in range(4):
                c = 4 * j + u
                wait_gather(u)
                fire_wb(c, u)
                nxt = (u + 3) % 4

                @pl.when((c >= 1) & (c + 3 < n_chunks))
                def _():
                    wait_wb(nxt)

                @pl.when(c + 3 < n_chunks)
                def _():
                    fire_gather(c + 3, nxt)
            return carry

        jax.lax.fori_loop(0, n_chunks // 4, quad, 0)
        for u in range(4):
            wait_wb(u)

    xsi = gather_kernel(xi, idx3)
    return xsi


def kernel(hidden_states, gate_w, w1, w2, w3):
    x = hidden_states.reshape(-1, HIDDEN)
    gate_wp = jnp.zeros((LANES, HIDDEN), jnp.float32).at[:NUM_EXPERTS].set(gate_w)
    idx_out, wts_out, xi, hist_out = _run_router(x, gate_wp)
    idx2 = idx_out[:, :TOP_K]
    wts2 = wts_out[:, :TOP_K]

    h8 = hist_out.reshape(T // _HTOK, LANES)[:, :NUM_EXPERTS]
    counts = jnp.sum(h8, axis=0)
    pcnt = ((counts + _GTM - 1) // _GTM) * _GTM
    poff = jnp.concatenate([jnp.zeros((1,), pcnt.dtype), jnp.cumsum(pcnt)[:-1]])
    cprefix = jnp.cumsum(h8, axis=0) - h8
    vbase = (poff[None, :] + cprefix).astype(jnp.int32)
    info16 = vbase.reshape(_SC_NW, 16, NUM_EXPERTS).transpose(0, 2, 1)
    info = jnp.zeros((_SC_NW, NUM_EXPERTS, LANES), jnp.int32).at[
        :, :, :16].set(info16)
    tile_start = jnp.arange(_n_tiles()) * _GTM
    poff_end = jnp.cumsum(pcnt)
    gid = jnp.clip(jnp.sum(tile_start[:, None] >= poff_end[None, :], axis=1),
                   0, NUM_EXPERTS - 1).astype(jnp.int32)

    eid_t = idx2.reshape(_SC_NW, 16, 64).transpose(0, 2, 1).reshape(
        _SC_NW, _SLOTS_W)
    sorted_tok, inv_t = _run_sc_binning(eid_t, info)
    inv = inv_t.reshape(_SC_NW, 64, 16).transpose(0, 2, 1).reshape(TOP_K * T)

    xs = _run_sc_gather(xi, sorted_tok)
    w1b = w1.astype(jnp.bfloat16)
    w3b = w3.astype(jnp.bfloat16)
    w2b = w2.astype(jnp.bfloat16)
    ys = _run_gmm(gid, xs, w1b, w3b, w2b)

    pos_a = inv[0::TOP_K]
    pos_b = inv[1::TOP_K]
    final = wts2[:, 0:1] * ys[pos_a] + wts2[:, 1:2] * ys[pos_b]
    return final.reshape(BATCH, SEQ, HIDDEN)
